# baseline (speedup 1.0000x reference)
import functools

import jax
import jax.numpy as jnp
from jax import lax
from jax.experimental import pallas as pl
from jax.experimental.pallas import tpu as pltpu
from jax.experimental.pallas import tpu_sc as plsc

BATCH = 16384
D = 32
NC = 2
NS = 16
NW = NC * NS
BPW = BATCH // NW
CHUNK = 128
NCH = BPW // CHUNK
L = 16
GROUPS = BPW // L


def _rsqrt(x):
    i = plsc.bitcast(x, jnp.int32)
    i = jnp.int32(0x5F3759DF) - (i >> 1)
    y = plsc.bitcast(i, jnp.float32)
    for _ in range(3):
        y = y * (jnp.float32(1.5) - jnp.float32(0.5) * x * y * y)
    return y


def _body(user_hbm, item_hbm, utab_hbm, itab_hbm, out_hbm,
          uidx, iidx, urows, irows, outv, usem, isem):
    wid = lax.axis_index("s") * NC + lax.axis_index("c")

    pltpu.sync_copy(user_hbm.at[wid], uidx)
    pltpu.sync_copy(item_hbm.at[wid], iidx)

    ucp = [pltpu.async_copy(utab_hbm.at[uidx.at[j]],
                            urows.at[pl.ds(j * CHUNK, CHUNK)], usem)
           for j in range(NCH)]
    icp = [pltpu.async_copy(itab_hbm.at[iidx.at[j]],
                            irows.at[pl.ds(j * CHUNK, CHUNK)], isem)
           for j in range(NCH)]
    for c in ucp + icp:
        c.wait()

    def step(g, carry):
        rows = lax.iota(jnp.int32, L) + g * L
        dot = jnp.zeros((L,), jnp.float32)
        n2u = jnp.zeros((L,), jnp.float32)
        n2v = jnp.zeros((L,), jnp.float32)
        for j in range(D):
            col = jnp.full((L,), j, jnp.int32)
            u = plsc.load_gather(urows, [rows, col])
            v = plsc.load_gather(irows, [rows, col])
            dot = dot + u * v
            n2u = n2u + u * u
            n2v = n2v + v * v
        r = (dot
             * _rsqrt(jnp.maximum(n2u, jnp.float32(1e-24)))
             * _rsqrt(jnp.maximum(n2v, jnp.float32(1e-24))))
        outv[pl.ds(g * L, L)] = r
        return carry

    lax.fori_loop(0, GROUPS, step, 0)

    pltpu.sync_copy(outv, out_hbm.at[pl.ds(wid * BPW, BPW)])


_cosine = functools.partial(
    pl.kernel,
    out_type=jax.ShapeDtypeStruct((BATCH,), jnp.float32),
    mesh=plsc.VectorSubcoreMesh(core_axis_name="c", subcore_axis_name="s"),
    compiler_params=pltpu.CompilerParams(
        needs_layout_passes=False, use_tc_tiling_on_sc=False),
    scratch_types=[
        pltpu.VMEM((NCH, CHUNK), jnp.int32),
        pltpu.VMEM((NCH, CHUNK), jnp.int32),
        pltpu.VMEM((BPW, D), jnp.float32),
        pltpu.VMEM((BPW, D), jnp.float32),
        pltpu.VMEM((BPW,), jnp.float32),
        pltpu.SemaphoreType.DMA,
        pltpu.SemaphoreType.DMA,
    ],
)(_body)


def kernel(user, item, user_table, item_table):
    u3 = user.astype(jnp.int32).reshape(NW, NCH, CHUNK)
    i3 = item.astype(jnp.int32).reshape(NW, NCH, CHUNK)
    return _cosine(u3, i3, user_table, item_table)
